# baseline (speedup 1.0000x reference)
import functools

import jax
import jax.numpy as jnp
from jax import lax
from jax.experimental import pallas as pl
from jax.experimental.pallas import tpu as pltpu
from jax.experimental.pallas import tpu_sc as plsc

B = 16384
F = 26
CARD = 100000
D = 16
R = F * CARD

NC = 2
NS = 16
NW = NC * NS
L = 16

B_PER_W = B // NW
CHUNK = 128
NCHUNK = B_PER_W // CHUNK
GROUPS = CHUNK // L


def _fm_body(x_ref, emb_ref, lin_ref, bias_ref, out_ref,
             idx_v, idxd_v, emb_g, lin_v, outbuf, bias_v,
             sem_x, sem_emb, sem_lin):
    wid = lax.axis_index("s") * NC + lax.axis_index("c")
    base = wid * B_PER_W

    pltpu.sync_copy(bias_ref, bias_v)

    def chunk_body(c, carry):
        cbase = base + c * CHUNK
        x_cps = [pltpu.async_copy(x_ref.at[pl.ds(f * B + cbase, CHUNK)],
                                  idx_v.at[f], sem_x) for f in range(F)]
        for cp in x_cps:
            cp.wait()

        for f in range(F):
            for j in range(GROUPS):
                sl = pl.ds(j * L, L)
                flat = idx_v[f, sl] + f * CARD
                idx_v[f, sl] = flat
                for d in range(D):
                    idxd_v[f, pl.ds(d * CHUNK + j * L, L)] = flat + d * R

        emb_cps = []
        lin_cps = []
        for f in range(F):
            emb_cps.append(pltpu.async_copy(
                emb_ref.at[idxd_v.at[f]],
                emb_g.at[pl.ds(f * D * CHUNK, D * CHUNK)], sem_emb))
            lin_cps.append(pltpu.async_copy(
                lin_ref.at[idx_v.at[f]],
                lin_v.at[pl.ds(f * CHUNK, CHUNK)], sem_lin))
        for cp in emb_cps:
            cp.wait()
        for cp in lin_cps:
            cp.wait()

        bias_vec = bias_v[...]

        def group_body(g, gcarry):
            boff = g * L
            s = [jnp.zeros((L,), jnp.float32) for _ in range(D)]
            q = [jnp.zeros((L,), jnp.float32) for _ in range(D)]
            lacc = jnp.zeros((L,), jnp.float32)
            for f in range(F):
                for d in range(D):
                    v = emb_g[pl.ds((f * D + d) * CHUNK + boff, L)]
                    s[d] = s[d] + v
                    q[d] = q[d] + v * v
                lacc = lacc + lin_v[pl.ds(f * CHUNK + boff, L)]
            inter = jnp.zeros((L,), jnp.float32)
            for d in range(D):
                inter = inter + (s[d] * s[d] - q[d])
            outbuf[pl.ds(boff, L)] = lacc + bias_vec + 0.5 * inter
            return gcarry

        lax.fori_loop(0, GROUPS, group_body, 0)
        pltpu.sync_copy(outbuf, out_ref.at[pl.ds(cbase, CHUNK)])
        return carry

    lax.fori_loop(0, NCHUNK, chunk_body, 0)


@jax.jit
def _fm(x, emb_table, lin2, lin_b):
    mesh = plsc.VectorSubcoreMesh(core_axis_name="c", subcore_axis_name="s")
    assert emb_table.shape == (D * R,)
    return pl.kernel(
        _fm_body,
        out_type=jax.ShapeDtypeStruct((B,), jnp.float32),
        mesh=mesh,
        compiler_params=pltpu.CompilerParams(
            needs_layout_passes=False, use_tc_tiling_on_sc=False),
        scratch_types=[
            pltpu.VMEM((F, CHUNK), jnp.int32),
            pltpu.VMEM((F, D * CHUNK), jnp.int32),
            pltpu.VMEM((F * D * CHUNK,), jnp.float32),
            pltpu.VMEM((F * CHUNK,), jnp.float32),
            pltpu.VMEM((CHUNK,), jnp.float32),
            pltpu.VMEM((L,), jnp.float32),
            pltpu.SemaphoreType.DMA,
            pltpu.SemaphoreType.DMA,
            pltpu.SemaphoreType.DMA,
        ],
    )(x, emb_table, lin2, lin_b)


def kernel(x, emb_table, lin_w, lin_b):
    bias16 = jnp.broadcast_to(lin_b, (L,))
    xf = x.T.reshape(F * B)
    emb1 = emb_table.T.reshape(D * R)
    out = _fm(xf, emb1, lin_w, bias16)
    return out.reshape(B, 1)


# speedup vs baseline: 1.0070x; 1.0070x over previous
import functools

import jax
import jax.numpy as jnp
from jax import lax
from jax.experimental import pallas as pl
from jax.experimental.pallas import tpu as pltpu
from jax.experimental.pallas import tpu_sc as plsc

B = 16384
F = 26
CARD = 100000
D = 16
R = F * CARD

NC = 2
NS = 16
NW = NC * NS
L = 16

B_PER_W = B // NW
CHUNK = 128
NCHUNK = B_PER_W // CHUNK
GROUPS = CHUNK // L


def _fm_body(x_ref, emb_ref, lin_ref, bias_ref, out_ref,
             idx_v, emb_g, lin_v, outbuf, bias_v,
             sem_x, sem_emb, sem_lin):
    wid = lax.axis_index("s") * NC + lax.axis_index("c")
    base = wid * B_PER_W

    pltpu.sync_copy(bias_ref, bias_v)

    def chunk_body(c, carry):
        cbase = base + c * CHUNK
        x_cps = [pltpu.async_copy(x_ref.at[pl.ds(f * B + cbase, CHUNK)],
                                  idx_v.at[f], sem_x) for f in range(F)]
        for cp in x_cps:
            cp.wait()

        for f in range(1, F):
            for j in range(GROUPS):
                sl = pl.ds(j * L, L)
                idx_v[f, sl] = idx_v[f, sl] + f * CARD

        emb_cps = []
        lin_cps = []
        for f in range(F):
            for d in range(D):
                emb_cps.append(pltpu.async_copy(
                    emb_ref.at[d].at[idx_v.at[f]],
                    emb_g.at[pl.ds((f * D + d) * CHUNK, CHUNK)], sem_emb))
            lin_cps.append(pltpu.async_copy(
                lin_ref.at[idx_v.at[f]],
                lin_v.at[pl.ds(f * CHUNK, CHUNK)], sem_lin))
        for cp in emb_cps:
            cp.wait()
        for cp in lin_cps:
            cp.wait()

        bias_vec = bias_v[...]

        def group_body(g, gcarry):
            boff = g * L
            s = [jnp.zeros((L,), jnp.float32) for _ in range(D)]
            q = [jnp.zeros((L,), jnp.float32) for _ in range(D)]
            lacc = jnp.zeros((L,), jnp.float32)
            for f in range(F):
                for d in range(D):
                    v = emb_g[pl.ds((f * D + d) * CHUNK + boff, L)]
                    s[d] = s[d] + v
                    q[d] = q[d] + v * v
                lacc = lacc + lin_v[pl.ds(f * CHUNK + boff, L)]
            inter = jnp.zeros((L,), jnp.float32)
            for d in range(D):
                inter = inter + (s[d] * s[d] - q[d])
            outbuf[pl.ds(boff, L)] = lacc + bias_vec + 0.5 * inter
            return gcarry

        lax.fori_loop(0, GROUPS, group_body, 0)
        pltpu.sync_copy(outbuf, out_ref.at[pl.ds(cbase, CHUNK)])
        return carry

    lax.fori_loop(0, NCHUNK, chunk_body, 0)


@jax.jit
def _fm(x, emb_table, lin2, lin_b):
    mesh = plsc.VectorSubcoreMesh(core_axis_name="c", subcore_axis_name="s")
    assert emb_table.shape == (D, R)
    return pl.kernel(
        _fm_body,
        out_type=jax.ShapeDtypeStruct((B,), jnp.float32),
        mesh=mesh,
        compiler_params=pltpu.CompilerParams(
            needs_layout_passes=False, use_tc_tiling_on_sc=False),
        scratch_types=[
            pltpu.VMEM((F, CHUNK), jnp.int32),
            pltpu.VMEM((F * D * CHUNK,), jnp.float32),
            pltpu.VMEM((F * CHUNK,), jnp.float32),
            pltpu.VMEM((CHUNK,), jnp.float32),
            pltpu.VMEM((L,), jnp.float32),
            pltpu.SemaphoreType.DMA,
            pltpu.SemaphoreType.DMA,
            pltpu.SemaphoreType.DMA,
        ],
    )(x, emb_table, lin2, lin_b)


def kernel(x, emb_table, lin_w, lin_b):
    bias16 = jnp.broadcast_to(lin_b, (L,))
    xf = x.T.reshape(F * B)
    out = _fm(xf, emb_table.T, lin_w, bias16)
    return out.reshape(B, 1)
